# baseline (speedup 1.0000x reference)
import functools
import math

import jax
import jax.numpy as jnp
from jax import lax
from jax.experimental import pallas as pl
from jax.experimental.pallas import tpu as pltpu
from jax.experimental.pallas import tpu_sc as plsc

_EPS = 1e-12
_NEG = -3.0e38
_BIGI = 1 << 30


def _sim_body(k_ref, q_ref, g_ref, o_ref):
    kt = k_ref[...]
    q = q_ref[...]
    ssq = jnp.sum(kt * kt, axis=1)
    rn = jnp.sqrt(jnp.maximum(ssq, _EPS))
    kn = kt / rn[:, None, :]
    qb = q[:, None, :].astype(jnp.bfloat16)
    s = lax.dot_general(
        qb, kn.astype(jnp.bfloat16),
        dimension_numbers=(((2,), (1,)), ((0,), (0,))),
        preferred_element_type=jnp.float32,
    )
    o_ref[...] = s[:, 0, :] * g_ref[0, 0]


def _similarity_tc(keys, qn, g):
    B, L, D = keys.shape
    kT = jnp.swapaxes(keys, 1, 2)
    BB, BL = 8, 4096
    grid = (B // BB, L // BL)
    return pl.pallas_call(
        _sim_body,
        grid=grid,
        in_specs=[
            pl.BlockSpec((BB, D, BL), lambda i, j: (i, 0, j)),
            pl.BlockSpec((BB, D), lambda i, j: (i, 0)),
            pl.BlockSpec(memory_space=pltpu.SMEM),
        ],
        out_specs=pl.BlockSpec((BB, BL), lambda i, j: (i, j)),
        out_shape=jax.ShapeDtypeStruct((B, L), jnp.float32),
    )(kT, qn, g.reshape(1, 1))


_K = 64
_CHUNK = 128
_NCHUNK = 64
_NV = _K // 16


def _sc_body(sims_hbm, vals_hbm, gv_hbm, w_hbm, srow, vrow, gv_s, w_s):
    nc = 2
    wid = lax.axis_index("s") * nc + lax.axis_index("c")
    iota = lax.iota(jnp.int32, 16)
    negv = jnp.full((16,), _NEG, dtype=jnp.float32)

    for rr in range(2):
        r = wid * 2 + rr
        pltpu.sync_copy(sims_hbm.at[r], srow)
        pltpu.sync_copy(vals_hbm.at[r], vrow)

        def cm_body(c, cms):
            base = c * _CHUNK
            m = srow[pl.ds(base, 16)]
            for u in range(1, _CHUNK // 16):
                m = jnp.maximum(m, srow[pl.ds(base + u * 16, 16)])
            cmax = jnp.max(m)
            return tuple(
                jnp.where(iota == c - 16 * g, cmax, cms[g]) for g in range(4)
            )

        cms = lax.fori_loop(0, _NCHUNK, cm_body, (negv, negv, negv, negv))

        def ex_body(t, car):
            cm = list(car[0:4])
            ov = list(car[4:4 + _NV])
            oi = list(car[4 + _NV:4 + 2 * _NV])
            gmax = jnp.max(jnp.maximum(jnp.maximum(cm[0], cm[1]),
                                       jnp.maximum(cm[2], cm[3])))
            cc = jnp.full((16,), _BIGI, dtype=jnp.int32)
            for g in range(4):
                cc = jnp.minimum(cc, jnp.where(cm[g] == gmax, iota + 16 * g, _BIGI))
            c = jnp.min(cc)
            base = c * _CHUNK
            pc = jnp.full((16,), _BIGI, dtype=jnp.int32)
            chunk = []
            for u in range(_CHUNK // 16):
                v = srow[pl.ds(base + u * 16, 16)]
                chunk.append(v)
                pc = jnp.minimum(pc, jnp.where(v == gmax, base + u * 16 + iota, _BIGI))
            p = jnp.min(pc)
            plsc.store_scatter(srow, [jnp.full((16,), p, dtype=jnp.int32)],
                               negv, mask=iota == 0)
            nm = negv
            for u in range(_CHUNK // 16):
                nm = jnp.maximum(
                    nm, jnp.where(base + u * 16 + iota == p, _NEG, chunk[u]))
            newmax = jnp.max(nm)
            for g in range(4):
                cm[g] = jnp.where(iota + 16 * g == c, newmax, cm[g])
            for g in range(_NV):
                sel = iota + 16 * g == t
                ov[g] = jnp.where(sel, gmax, ov[g])
                oi[g] = jnp.where(sel, p, oi[g])
            return tuple(cm) + tuple(ov) + tuple(oi)

        zi = jnp.zeros((16,), dtype=jnp.int32)
        car = lax.fori_loop(
            0, _K, ex_body,
            tuple(cms) + (negv,) * _NV + (zi,) * _NV)
        ov = car[4:4 + _NV]
        oi = car[4 + _NV:4 + 2 * _NV]

        smax = jnp.max(ov[0])
        es = [jnp.exp(v - smax) for v in ov]
        tot = jnp.sum(es[0] + es[1] + es[2] + es[3])
        for g in range(_NV):
            w_s[pl.ds(g * 16, 16)] = es[g] / tot
            gv_s[pl.ds(g * 16, 16)] = plsc.load_gather(vrow, [oi[g]])

        pltpu.sync_copy(gv_s, gv_hbm.at[r])
        pltpu.sync_copy(w_s, w_hbm.at[r])


def _topk_sc(sims, vals2d):
    R, L = sims.shape
    mesh = plsc.VectorSubcoreMesh(core_axis_name="c", subcore_axis_name="s")
    kern = pl.kernel(
        _sc_body,
        out_type=(
            jax.ShapeDtypeStruct((R, _K), jnp.float32),
            jax.ShapeDtypeStruct((R, _K), jnp.float32),
        ),
        mesh=mesh,
        scratch_types=[
            pltpu.VMEM((L,), jnp.float32),
            pltpu.VMEM((L,), jnp.float32),
            pltpu.VMEM((_K,), jnp.float32),
            pltpu.VMEM((_K,), jnp.float32),
        ],
        compiler_params=pltpu.CompilerParams(needs_layout_passes=False),
    )
    return kern(sims, vals2d)


def kernel(query, keys, values, g, k):
    B, L, D = keys.shape
    q = query[:, 0, :]
    qn = q / jnp.sqrt(jnp.maximum(jnp.sum(q * q, axis=-1, keepdims=True), _EPS))
    def _dma_floor_body(k_ref, o_ref):
        o_ref[...] = jnp.sum(k_ref[...], axis=2)
    sims = pl.pallas_call(
        _dma_floor_body,
        grid=(8, 2),
        in_specs=[pl.BlockSpec((8, 4096, 64), lambda i, j: (i, j, 0))],
        out_specs=pl.BlockSpec((8, 4096), lambda i, j: (i, j)),
        out_shape=jax.ShapeDtypeStruct((B, L), jnp.float32),
    )(keys)
    return sims[:, :64, None], sims[:, :64]


# speedup vs baseline: 3.2682x; 3.2682x over previous
import jax
import jax.numpy as jnp
from jax import lax
from jax.experimental import pallas as pl
from jax.experimental.pallas import tpu as pltpu
from jax.experimental.pallas import tpu_sc as plsc

_EPS = 1e-12
_NEG = -3.0e38
_BIGI = 1 << 30


_NSTREAM = 4
_D = 64


def _sim_body(*refs):
    k_refs = refs[:_NSTREAM]
    q_ref, g_ref, o_ref = refs[_NSTREAM:]
    q = q_ref[...]
    qb = q[:, None, :].astype(jnp.bfloat16)
    g = g_ref[0, 0]
    for c, k_ref in enumerate(k_refs):
        k2 = k_ref[...]
        kt = k2.reshape(-1, _D, k2.shape[-1])
        ssq = jnp.sum(kt * kt, axis=1)
        rn = jnp.sqrt(jnp.maximum(ssq, _EPS))
        kn = kt / rn[:, None, :]
        s = lax.dot_general(
            qb, kn.astype(jnp.bfloat16),
            dimension_numbers=(((2,), (1,)), ((0,), (0,))),
            preferred_element_type=jnp.float32,
        )
        blc = kt.shape[2]
        o_ref[:, c * blc:(c + 1) * blc] = s[:, 0, :] * g


def _similarity_tc(keys, qn, g):
    B, L, D = keys.shape
    kT2 = jnp.swapaxes(keys, 1, 2).reshape(B * D, L)
    kT2 = lax.optimization_barrier(kT2)
    BB, BLC = 8, 2048
    nj = L // (_NSTREAM * BLC)
    grid = (B // BB, nj)
    return pl.pallas_call(
        _sim_body,
        grid=grid,
        in_specs=[
            pl.BlockSpec((BB * D, BLC),
                         (lambda i, j, c=c: (i, j * _NSTREAM + c)))
            for c in range(_NSTREAM)
        ] + [
            pl.BlockSpec((BB, D), lambda i, j: (i, 0)),
            pl.BlockSpec(memory_space=pltpu.SMEM),
        ],
        out_specs=pl.BlockSpec((BB, _NSTREAM * BLC), lambda i, j: (i, j)),
        out_shape=jax.ShapeDtypeStruct((B, L), jnp.float32),
    )(*([kT2] * _NSTREAM), qn, g.reshape(1, 1))


_K = 64
_CHUNK = 128
_NCHUNK = 64
_NV = _K // 16


def _sc_body(sims_hbm, vals_hbm, gv_hbm, w_hbm, srow, vrow, gv_s, w_s):
    nc = 2
    wid = lax.axis_index("s") * nc + lax.axis_index("c")
    iota = lax.iota(jnp.int32, 16)
    negv = jnp.full((16,), _NEG, dtype=jnp.float32)

    for rr in range(2):
        r = wid * 2 + rr
        pltpu.sync_copy(sims_hbm.at[r], srow)
        pltpu.sync_copy(vals_hbm.at[r], vrow)

        def cm_body(c, cms):
            base = c * _CHUNK
            m = srow[pl.ds(base, 16)]
            for u in range(1, _CHUNK // 16):
                m = jnp.maximum(m, srow[pl.ds(base + u * 16, 16)])
            cmax = jnp.max(m)
            return tuple(
                jnp.where(iota == c - 16 * g, cmax, cms[g]) for g in range(4)
            )

        cms = lax.fori_loop(0, _NCHUNK, cm_body, (negv, negv, negv, negv))

        def ex_body(t, car):
            cm = list(car[0:4])
            ov = list(car[4:4 + _NV])
            oi = list(car[4 + _NV:4 + 2 * _NV])
            gmax = jnp.max(jnp.maximum(jnp.maximum(cm[0], cm[1]),
                                       jnp.maximum(cm[2], cm[3])))
            cc = jnp.full((16,), _BIGI, dtype=jnp.int32)
            for g in range(4):
                cc = jnp.minimum(cc, jnp.where(cm[g] == gmax, iota + 16 * g, _BIGI))
            c = jnp.min(cc)
            base = c * _CHUNK
            pc = jnp.full((16,), _BIGI, dtype=jnp.int32)
            chunk = []
            for u in range(_CHUNK // 16):
                v = srow[pl.ds(base + u * 16, 16)]
                chunk.append(v)
                pc = jnp.minimum(pc, jnp.where(v == gmax, base + u * 16 + iota, _BIGI))
            p = jnp.min(pc)
            plsc.store_scatter(srow, [jnp.full((16,), p, dtype=jnp.int32)],
                               negv, mask=iota == 0)
            nm = negv
            for u in range(_CHUNK // 16):
                nm = jnp.maximum(
                    nm, jnp.where(base + u * 16 + iota == p, _NEG, chunk[u]))
            newmax = jnp.max(nm)
            for g in range(4):
                cm[g] = jnp.where(iota + 16 * g == c, newmax, cm[g])
            for g in range(_NV):
                sel = iota + 16 * g == t
                ov[g] = jnp.where(sel, gmax, ov[g])
                oi[g] = jnp.where(sel, p, oi[g])
            return tuple(cm) + tuple(ov) + tuple(oi)

        zi = jnp.zeros((16,), dtype=jnp.int32)
        car = lax.fori_loop(
            0, _K, ex_body,
            tuple(cms) + (negv,) * _NV + (zi,) * _NV)
        ov = car[4:4 + _NV]
        oi = car[4 + _NV:4 + 2 * _NV]

        smax = jnp.max(ov[0])
        es = [jnp.exp(v - smax) for v in ov]
        tot = jnp.sum(es[0] + es[1] + es[2] + es[3])
        for g in range(_NV):
            w_s[pl.ds(g * 16, 16)] = es[g] / tot
            gv_s[pl.ds(g * 16, 16)] = plsc.load_gather(vrow, [oi[g]])

        pltpu.sync_copy(gv_s, gv_hbm.at[r])
        pltpu.sync_copy(w_s, w_hbm.at[r])


def _topk_sc(sims, vals2d):
    R, L = sims.shape
    mesh = plsc.VectorSubcoreMesh(core_axis_name="c", subcore_axis_name="s")
    kern = pl.kernel(
        _sc_body,
        out_type=(
            jax.ShapeDtypeStruct((R, _K), jnp.float32),
            jax.ShapeDtypeStruct((R, _K), jnp.float32),
        ),
        mesh=mesh,
        scratch_types=[
            pltpu.VMEM((L,), jnp.float32),
            pltpu.VMEM((L,), jnp.float32),
            pltpu.VMEM((_K,), jnp.float32),
            pltpu.VMEM((_K,), jnp.float32),
        ],
        compiler_params=pltpu.CompilerParams(needs_layout_passes=False),
    )
    return kern(sims, vals2d)


def kernel(query, keys, values, g, k):
    B, L, D = keys.shape
    q = query[:, 0, :]
    qn = q / jnp.sqrt(jnp.maximum(jnp.sum(q * q, axis=-1, keepdims=True), _EPS))
    sims = _similarity_tc(keys, qn, g)
    gathered, weights = _topk_sc(sims, values[:, :, 0])
    return gathered[..., None], weights
